# baseline (speedup 1.0000x reference)
import jax
import jax.numpy as jnp
from jax import lax
from jax.experimental import pallas as pl

NUM_CLASSES = 1000
BLOCK_ROWS = 128


def _onehot_body(x_ref, out_ref):
    idx = x_ref[...].astype(jnp.int32)
    classes = lax.broadcasted_iota(jnp.int32, (BLOCK_ROWS, 20, NUM_CLASSES), 2)
    out_ref[...] = (idx[:, :, None] == classes).astype(jnp.float32)


def kernel(x):
    B, S = x.shape
    grid = (B // BLOCK_ROWS,)
    return pl.pallas_call(
        _onehot_body,
        grid=grid,
        in_specs=[pl.BlockSpec((BLOCK_ROWS, S), lambda i: (i, 0))],
        out_specs=pl.BlockSpec((BLOCK_ROWS, S, NUM_CLASSES), lambda i: (i, 0, 0)),
        out_shape=jax.ShapeDtypeStruct((B, S, NUM_CLASSES), jnp.float32),
    )(x.astype(jnp.int32))
